# baseline (speedup 1.0000x reference)
import functools

import jax
import jax.numpy as jnp
from jax.experimental import pallas as pl
from jax.experimental.pallas import tpu as pltpu

_V = 30000
_NCORES = 2
_NCHUNK = 8


def _round_up(x, m):
    return ((x + m - 1) // m) * m


def _pool_head_kernel(ids_ref, emb_hbm, w_ref, b_ref, o_ref,
                      emb_vmem, pooled_ref, sems, *, rows_per_core, s):
    nmaj, _, hp = emb_vmem.shape
    ch = nmaj // _NCHUNK

    def chunk_copy(k):
        return pltpu.make_async_copy(emb_hbm.at[pl.ds(k * ch, ch)],
                                     emb_vmem.at[pl.ds(k * ch, ch)],
                                     sems.at[k])
    for k in range(_NCHUNK):
        chunk_copy(k).start()
    for k in range(_NCHUNK):
        chunk_copy(k).wait()

    row0 = pl.program_id(0) * rows_per_core
    iota8 = jax.lax.broadcasted_iota(jnp.int32, (8, hp), 0)

    def group(i, carry):
        accs = []
        for rr in range(8):
            row = row0 + i * 8 + rr
            acc8 = None
            for t in range(s):
                tok = ids_ref[row, t]
                chunk = emb_vmem[jax.lax.shift_right_logical(tok, 3)]
                rolled = pltpu.roll(chunk, -jnp.bitwise_and(tok, 7), axis=0)
                acc8 = rolled if acc8 is None else acc8 + rolled
            accs.append(acc8)
        tile = jnp.zeros_like(accs[0])
        for rr in range(8):
            tile = jnp.where(iota8 == rr, pltpu.roll(accs[rr], rr, axis=0),
                             tile)
        pooled_ref[pl.ds(pl.multiple_of(i * 8, 8), 8), :] = tile
        return carry

    jax.lax.fori_loop(0, rows_per_core // 8, group, 0)

    logits = jnp.dot(pooled_ref[...], w_ref[...],
                     preferred_element_type=jnp.float32)
    o_ref[...] = logits + b_ref[...]


def kernel(x, emb, w, b):
    B, S = x.shape
    Vr, Hp = emb.shape
    Cp = w.shape[1]
    Bp = _round_up(B, 8 * _NCORES)
    rows_per_core = Bp // _NCORES

    ids = x.astype(jnp.int32)
    ids = jnp.where((ids >= 0) & (ids < _V), ids, _V)
    ids = jnp.pad(ids, ((0, Bp - B), (0, 0)), constant_values=_V)

    emb3 = emb.reshape(Vr // 8, 8, Hp)
    w_scaled = w * jnp.float32(1.0 / S)

    out = pl.pallas_call(
        functools.partial(_pool_head_kernel, rows_per_core=rows_per_core, s=S),
        out_shape=jax.ShapeDtypeStruct((Bp, Cp), jnp.float32),
        grid_spec=pltpu.PrefetchScalarGridSpec(
            num_scalar_prefetch=1,
            grid=(_NCORES,),
            in_specs=[
                pl.BlockSpec(memory_space=pl.ANY),
                pl.BlockSpec((Hp, Cp), lambda i, ids: (0, 0),
                             pipeline_mode=pl.Buffered(1)),
                pl.BlockSpec((1, Cp), lambda i, ids: (0, 0),
                             pipeline_mode=pl.Buffered(1)),
            ],
            out_specs=pl.BlockSpec((rows_per_core, Cp), lambda i, ids: (i, 0)),
            scratch_shapes=[
                pltpu.VMEM((Vr // 8, 8, Hp), jnp.float32),
                pltpu.VMEM((rows_per_core, Hp), jnp.float32),
                pltpu.SemaphoreType.DMA((_NCHUNK,)),
            ],
        ),
        compiler_params=pltpu.CompilerParams(
            dimension_semantics=("parallel",),
            vmem_limit_bytes=48 * 1024 * 1024,
        ),
    )(ids, emb3, w_scaled, b)
    return out[:B, :Cp]


# speedup vs baseline: 1.7308x; 1.7308x over previous
import functools

import jax
import jax.numpy as jnp
from jax.experimental import pallas as pl
from jax.experimental.pallas import tpu as pltpu

_V = 30000


def _round_up(x, m):
    return ((x + m - 1) // m) * m


def _pool_head_kernel(base_ref, rot_ref, emb_ref, w_ref, b_ref, o_ref,
                      pooled_ref, *, tb, s):
    row0 = pl.program_id(0) * tb
    for r in range(tb):
        acc8 = None
        for t in range(s):
            row = row0 + r
            chunk = emb_ref[pl.ds(pl.multiple_of(base_ref[row, t], 8), 8), :]
            rolled = pltpu.roll(chunk, rot_ref[row, t], axis=0)
            acc8 = rolled if acc8 is None else acc8 + rolled
        pooled_ref[r, :] = acc8[0, :]
    logits = jnp.dot(pooled_ref[...], w_ref[...],
                     preferred_element_type=jnp.float32)
    o_ref[...] = logits + b_ref[...]


def kernel(x, emb, w, b):
    B, S = x.shape
    Vr, Hp = emb.shape
    Cp = w.shape[1]
    tb = 8
    Bp = _round_up(B, tb)

    ids = x.astype(jnp.int32)
    ids = jnp.where((ids >= 0) & (ids < _V), ids, _V)
    ids = jnp.pad(ids, ((0, Bp - B), (0, 0)), constant_values=_V)
    ids_base = ids & ~jnp.int32(7)
    ids_rot = (-ids) & jnp.int32(7)

    w_scaled = w * jnp.float32(1.0 / S)

    out = pl.pallas_call(
        functools.partial(_pool_head_kernel, tb=tb, s=S),
        out_shape=jax.ShapeDtypeStruct((Bp, Cp), jnp.float32),
        grid_spec=pltpu.PrefetchScalarGridSpec(
            num_scalar_prefetch=2,
            grid=(Bp // tb,),
            in_specs=[
                pl.BlockSpec((Vr, Hp), lambda i, bs, rt: (0, 0),
                             pipeline_mode=pl.Buffered(1)),
                pl.BlockSpec((Hp, Cp), lambda i, bs, rt: (0, 0),
                             pipeline_mode=pl.Buffered(1)),
                pl.BlockSpec((1, Cp), lambda i, bs, rt: (0, 0),
                             pipeline_mode=pl.Buffered(1)),
            ],
            out_specs=pl.BlockSpec((tb, Cp), lambda i, bs, rt: (i, 0)),
            scratch_shapes=[pltpu.VMEM((tb, Hp), jnp.float32)],
        ),
        compiler_params=pltpu.CompilerParams(
            dimension_semantics=("parallel",),
            vmem_limit_bytes=48 * 1024 * 1024,
        ),
    )(ids_base, ids_rot, emb, w_scaled, b)
    return out[:B, :Cp]


# speedup vs baseline: 2.6997x; 1.5598x over previous
import functools

import jax
import jax.numpy as jnp
from jax.experimental import pallas as pl
from jax.experimental.pallas import tpu as pltpu

_V = 30000


def _round_up(x, m):
    return ((x + m - 1) // m) * m


def _pool_head_kernel(idx_ref, emb_ref, w_ref, b_ref, o_ref, pooled_ref,
                      *, tb, s):
    row0 = pl.program_id(0) * tb
    for r in range(tb):
        acc2 = None
        for t in range(s):
            slab = emb_ref[pl.ds(pl.multiple_of(idx_ref[row0 + r, t], 2), 2), :]
            acc2 = slab if acc2 is None else acc2 + slab
        pooled_ref[pl.ds(r, 1), :] = jnp.concatenate(
            [acc2[0:1, :], acc2[1:2, :]], axis=1)
    logits = jnp.dot(pooled_ref[...], w_ref[...],
                     preferred_element_type=jnp.float32)
    o_ref[...] = logits + b_ref[...]


def kernel(x, emb, w, b):
    B, S = x.shape
    Vr, Hp = emb.shape
    Cp = w.shape[1]
    tb = 16
    Bp = _round_up(B, tb)

    ids = x.astype(jnp.int32)
    ids = jnp.where((ids >= 0) & (ids < _V), ids, _V)
    ids = jnp.pad(ids, ((0, Bp - B), (0, 0)), constant_values=_V)
    ids2 = ids * jnp.int32(2)

    emb2 = emb.reshape(2 * Vr, Hp // 2)
    w_scaled = w * jnp.float32(1.0 / S)

    out = pl.pallas_call(
        functools.partial(_pool_head_kernel, tb=tb, s=S),
        out_shape=jax.ShapeDtypeStruct((Bp, Cp), jnp.float32),
        grid_spec=pltpu.PrefetchScalarGridSpec(
            num_scalar_prefetch=1,
            grid=(Bp // tb,),
            in_specs=[
                pl.BlockSpec((2 * Vr, Hp // 2), lambda i, idx: (0, 0),
                             pipeline_mode=pl.Buffered(1)),
                pl.BlockSpec((Hp, Cp), lambda i, idx: (0, 0),
                             pipeline_mode=pl.Buffered(1)),
                pl.BlockSpec((1, Cp), lambda i, idx: (0, 0),
                             pipeline_mode=pl.Buffered(1)),
            ],
            out_specs=pl.BlockSpec((tb, Cp), lambda i, idx: (i, 0)),
            scratch_shapes=[pltpu.VMEM((tb, Hp), jnp.float32)],
        ),
        compiler_params=pltpu.CompilerParams(
            dimension_semantics=("parallel",),
            vmem_limit_bytes=48 * 1024 * 1024,
        ),
    )(ids2, emb2, w_scaled, b)
    return out[:B, :Cp]


# speedup vs baseline: 2.7502x; 1.0187x over previous
import functools

import jax
import jax.numpy as jnp
from jax.experimental import pallas as pl
from jax.experimental.pallas import tpu as pltpu

_V = 30000


def _round_up(x, m):
    return ((x + m - 1) // m) * m


def _pool_head_kernel(idx_ref, emb_ref, w_ref, b_ref, o_ref, pooled_ref,
                      *, tb, s):
    row0 = pl.program_id(0) * tb
    for r in range(tb):
        acc2 = None
        for t in range(s):
            slab = emb_ref[pl.ds(pl.multiple_of(idx_ref[row0 + r, t], 2), 2), :]
            acc2 = slab if acc2 is None else acc2 + slab
        pooled_ref[pl.ds(r, 1), :] = jnp.concatenate(
            [acc2[0:1, :], acc2[1:2, :]], axis=1)
    logits = jnp.dot(pooled_ref[...], w_ref[...],
                     preferred_element_type=jnp.float32)
    o_ref[...] = logits + b_ref[...]


def kernel(x, emb, w, b):
    B, S = x.shape
    Vr, Hp = emb.shape
    Cp = w.shape[1]
    tb = 32
    Bp = _round_up(B, tb)

    ids = x.astype(jnp.int32)
    ids = jnp.where((ids >= 0) & (ids < _V), ids, _V)
    ids = jnp.pad(ids, ((0, Bp - B), (0, 0)), constant_values=_V)
    ids2 = ids * jnp.int32(2)

    emb2 = emb.reshape(2 * Vr, Hp // 2)
    w_scaled = w * jnp.float32(1.0 / S)

    out = pl.pallas_call(
        functools.partial(_pool_head_kernel, tb=tb, s=S),
        out_shape=jax.ShapeDtypeStruct((Bp, Cp), jnp.float32),
        grid_spec=pltpu.PrefetchScalarGridSpec(
            num_scalar_prefetch=1,
            grid=(Bp // tb,),
            in_specs=[
                pl.BlockSpec((2 * Vr, Hp // 2), lambda i, idx: (0, 0),
                             pipeline_mode=pl.Buffered(1)),
                pl.BlockSpec((Hp, Cp), lambda i, idx: (0, 0),
                             pipeline_mode=pl.Buffered(1)),
                pl.BlockSpec((1, Cp), lambda i, idx: (0, 0),
                             pipeline_mode=pl.Buffered(1)),
            ],
            out_specs=pl.BlockSpec((tb, Cp), lambda i, idx: (i, 0)),
            scratch_shapes=[pltpu.VMEM((tb, Hp), jnp.float32)],
        ),
        compiler_params=pltpu.CompilerParams(
            dimension_semantics=("parallel",),
            vmem_limit_bytes=48 * 1024 * 1024,
        ),
    )(ids2, emb2, w_scaled, b)
    return out[:B, :Cp]


# speedup vs baseline: 2.7941x; 1.0160x over previous
import functools

import jax
import jax.numpy as jnp
from jax.experimental import pallas as pl
from jax.experimental.pallas import tpu as pltpu

_V = 30000


def _round_up(x, m):
    return ((x + m - 1) // m) * m


def _pool_head_kernel(idx_ref, emb_ref, w_ref, b_ref, o_ref, pooled_ref,
                      *, tb, s):
    row0 = pl.program_id(0) * tb
    for r in range(tb):
        acc2 = None
        for t in range(s):
            slab = emb_ref[pl.ds(pl.multiple_of(idx_ref[row0 + r, t], 2), 2), :]
            acc2 = slab if acc2 is None else acc2 + slab
        pooled_ref[pl.ds(r, 1), :] = jnp.concatenate(
            [acc2[0:1, :], acc2[1:2, :]], axis=1)
    logits = jnp.dot(pooled_ref[...], w_ref[...],
                     preferred_element_type=jnp.float32)
    o_ref[...] = logits + b_ref[...]


def kernel(x, emb, w, b):
    B, S = x.shape
    Vr, Hp = emb.shape
    Cp = w.shape[1]
    tb = 64
    Bp = _round_up(B, tb)

    ids = x.astype(jnp.int32)
    ids = jnp.where((ids >= 0) & (ids < _V), ids, _V)
    ids = jnp.pad(ids, ((0, Bp - B), (0, 0)), constant_values=_V)
    ids2 = ids * jnp.int32(2)

    emb2 = emb.reshape(2 * Vr, Hp // 2)
    w_scaled = w * jnp.float32(1.0 / S)

    out = pl.pallas_call(
        functools.partial(_pool_head_kernel, tb=tb, s=S),
        out_shape=jax.ShapeDtypeStruct((Bp, Cp), jnp.float32),
        grid_spec=pltpu.PrefetchScalarGridSpec(
            num_scalar_prefetch=1,
            grid=(Bp // tb,),
            in_specs=[
                pl.BlockSpec((2 * Vr, Hp // 2), lambda i, idx: (0, 0),
                             pipeline_mode=pl.Buffered(1)),
                pl.BlockSpec((Hp, Cp), lambda i, idx: (0, 0),
                             pipeline_mode=pl.Buffered(1)),
                pl.BlockSpec((1, Cp), lambda i, idx: (0, 0),
                             pipeline_mode=pl.Buffered(1)),
            ],
            out_specs=pl.BlockSpec((tb, Cp), lambda i, idx: (i, 0)),
            scratch_shapes=[pltpu.VMEM((tb, Hp), jnp.float32)],
        ),
        compiler_params=pltpu.CompilerParams(
            dimension_semantics=("parallel",),
            vmem_limit_bytes=48 * 1024 * 1024,
        ),
    )(ids2, emb2, w_scaled, b)
    return out[:B, :Cp]


# speedup vs baseline: 2.9301x; 1.0487x over previous
import functools

import jax
import jax.numpy as jnp
from jax.experimental import pallas as pl
from jax.experimental.pallas import tpu as pltpu

_V = 30000


def _round_up(x, m):
    return ((x + m - 1) // m) * m


def _pool_head_kernel(idx_ref, emb_ref, w_ref, b_ref, o_ref, pooled_ref,
                      *, tb, s):
    row0 = pl.program_id(0) * tb
    for r in range(tb):
        acc_a, acc_b = None, None
        for t in range(s):
            slab = emb_ref[pl.ds(pl.multiple_of(idx_ref[row0 + r, t], 2), 2), :]
            if t % 2 == 0:
                acc_a = slab if acc_a is None else acc_a + slab
            else:
                acc_b = slab if acc_b is None else acc_b + slab
        acc2 = acc_a if acc_b is None else acc_a + acc_b
        pooled_ref[pl.ds(r, 1), :] = jnp.concatenate(
            [acc2[0:1, :], acc2[1:2, :]], axis=1)
    logits = jnp.dot(pooled_ref[...], w_ref[...],
                     preferred_element_type=jnp.float32)
    o_ref[...] = logits + b_ref[...]


def kernel(x, emb, w, b):
    B, S = x.shape
    Vr, Hp = emb.shape
    Cp = w.shape[1]
    tb = 64
    Bp = _round_up(B, tb)

    ids = x.astype(jnp.int32)
    ids = jnp.where((ids >= 0) & (ids < _V), ids, _V)
    ids = jnp.pad(ids, ((0, Bp - B), (0, 0)), constant_values=_V)
    ids2 = ids * jnp.int32(2)

    emb2 = emb.reshape(2 * Vr, Hp // 2)
    w_scaled = w * jnp.float32(1.0 / S)

    out = pl.pallas_call(
        functools.partial(_pool_head_kernel, tb=tb, s=S),
        out_shape=jax.ShapeDtypeStruct((Bp, Cp), jnp.float32),
        grid_spec=pltpu.PrefetchScalarGridSpec(
            num_scalar_prefetch=1,
            grid=(Bp // tb,),
            in_specs=[
                pl.BlockSpec((2 * Vr, Hp // 2), lambda i, idx: (0, 0),
                             pipeline_mode=pl.Buffered(1)),
                pl.BlockSpec((Hp, Cp), lambda i, idx: (0, 0),
                             pipeline_mode=pl.Buffered(1)),
                pl.BlockSpec((1, Cp), lambda i, idx: (0, 0),
                             pipeline_mode=pl.Buffered(1)),
            ],
            out_specs=pl.BlockSpec((tb, Cp), lambda i, idx: (i, 0)),
            scratch_shapes=[pltpu.VMEM((tb, Hp), jnp.float32)],
        ),
        compiler_params=pltpu.CompilerParams(
            dimension_semantics=("parallel",),
            vmem_limit_bytes=48 * 1024 * 1024,
        ),
    )(ids2, emb2, w_scaled, b)
    return out[:B, :Cp]
